# baseline (speedup 1.0000x reference)
import functools

import jax
import jax.numpy as jnp
from jax import lax
from jax.experimental import pallas as pl
from jax.experimental.pallas import tpu as pltpu
from jax.experimental.pallas import tpu_sc as plsc

N = 10000
E = 320000
D = 128
NC, NS, L = 2, 16, 16
K = 80
E_TILE = E // (NC * NS)
MAIN_CHUNKS = E_TILE // K
NBUF = 3
NIDX = 4
DGB = 5
DEG_BLOCKS = MAIN_CHUNKS // DGB
NSTRIPE = 640
LAST_STRIPE = N - NSTRIPE * (NS - 1)

_SC_PARAMS = pltpu.CompilerParams(needs_layout_passes=False,
                                  use_tc_tiling_on_sc=False)
_SC_MESH = plsc.VectorSubcoreMesh(core_axis_name="c", subcore_axis_name="s")


def _deg_body(dstm_hbm, ewm_hbm, degp_hbm,
              deg_sh, ddst, dewv, z1d, sem_dma, sems_deg):
    c = lax.axis_index("c")
    s = lax.axis_index("s")
    t = c * NS + s
    zero16 = jnp.zeros((L,), jnp.float32)

    @pl.loop(0, 1040 // L)
    def _(i):
        z1d[pl.ds(i * L, L)] = zero16

    @pl.when(s == 0)
    def _():
        for k in range(N // 1000):
            pltpu.sync_copy(z1d.at[pl.ds(0, 1000)],
                            deg_sh.at[pl.ds(k * 1000, 1000)])

    plsc.subcore_barrier()

    def issue_dma(blk, q):
        pltpu.async_copy(dstm_hbm.at[t].at[pl.ds(blk * DGB, DGB)],
                         ddst[q], sem_dma)
        pltpu.async_copy(ewm_hbm.at[t].at[pl.ds(blk * DGB, DGB)],
                         dewv[q], sem_dma)

    def wait_dma(blk, q):
        pltpu.make_async_copy(dstm_hbm.at[t].at[pl.ds(blk * DGB, DGB)],
                              ddst[q], sem_dma).wait()
        pltpu.make_async_copy(ewm_hbm.at[t].at[pl.ds(blk * DGB, DGB)],
                              dewv[q], sem_dma).wait()

    def process(q):
        @pl.loop(0, DGB)
        def _(r):
            for g in range(K // L):
                sl = pl.ds(g * L, L)
                dewv[q][r, sl] = jnp.maximum(dewv[q][r, sl], 1e-5)

        for r in range(DGB):
            pltpu.async_copy(dewv[q].at[r], deg_sh.at[ddst[q].at[r]],
                             sems_deg[q], add=True)

    def drain(q):
        for r in range(DGB):
            pltpu.make_async_copy(dewv[q].at[r], deg_sh.at[ddst[q].at[r]],
                                  sems_deg[q]).wait()

    issue_dma(0, 0)
    for blk in range(DEG_BLOCKS):
        q = blk % 2
        wait_dma(blk, q)
        if blk > 0:
            drain(1 - q)
        if blk + 1 < DEG_BLOCKS:
            issue_dma(blk + 1, 1 - q)
        process(q)
    drain((DEG_BLOCKS - 1) % 2)

    plsc.subcore_barrier()

    @pl.when(s < NS - 1)
    def _():
        pltpu.sync_copy(deg_sh.at[pl.ds(s * NSTRIPE, NSTRIPE)],
                        degp_hbm.at[c, pl.ds(s * NSTRIPE, NSTRIPE)])

    @pl.when(s == NS - 1)
    def _():
        base = (NS - 1) * NSTRIPE
        pltpu.sync_copy(deg_sh.at[pl.ds(base, LAST_STRIPE)],
                        degp_hbm.at[c, pl.ds(base, LAST_STRIPE)])


_sc_deg = functools.partial(
    pl.kernel,
    out_type=jax.ShapeDtypeStruct((NC, N), jnp.float32),
    mesh=_SC_MESH,
    compiler_params=_SC_PARAMS,
    scratch_types=[
        pltpu.VMEM_SHARED((N,), jnp.float32),
        [pltpu.VMEM((DGB, K), jnp.int32)] * 2,
        [pltpu.VMEM((DGB, K), jnp.float32)] * 2,
        pltpu.VMEM((1040,), jnp.float32),
        pltpu.SemaphoreType.DMA,
        [pltpu.SemaphoreType.DMA] * 2,
    ],
)(_deg_body)


def _sc_body(srcm_hbm, dstm_hbm, ewm_hbm, h_hbm, dinv_hbm, part_hbm,
             acc_sh, sidx, didx, coefb, dinv, rowbufs,
             sem_dinv, sems_i, sems_g, sems_s):
    c = lax.axis_index("c")
    s = lax.axis_index("s")
    t = c * NS + s
    zero16 = jnp.zeros((L,), jnp.float32)

    dinv_cp = pltpu.async_copy(dinv_hbm, dinv, sem_dinv)

    zrow = rowbufs[0]

    @pl.loop(0, K)
    def _(r):
        for k in range(D // L):
            zrow[r, pl.ds(k * L, L)] = zero16

    @pl.when(s < NS - 1)
    def _():
        for k in range(NSTRIPE // K):
            pltpu.sync_copy(zrow, acc_sh.at[pl.ds(s * NSTRIPE + k * K, K)])

    @pl.when(s == NS - 1)
    def _():
        for k in range(LAST_STRIPE // K):
            pltpu.sync_copy(zrow,
                            acc_sh.at[pl.ds((NS - 1) * NSTRIPE + k * K, K)])

    plsc.subcore_barrier()

    def issue_idx(i, p):
        pltpu.async_copy(srcm_hbm.at[t].at[i], sidx[p], sems_i[p])
        pltpu.async_copy(dstm_hbm.at[t].at[i], didx[p], sems_i[p])
        pltpu.async_copy(ewm_hbm.at[t].at[i], coefb[p], sems_i[p])

    def wait_idx(i, p):
        for _ in range(3):
            pltpu.make_async_copy(srcm_hbm.at[t].at[i], sidx[p],
                                  sems_i[p]).wait()

    def issue_gather(pb, pi):
        pltpu.async_copy(h_hbm.at[sidx[pi]], rowbufs[pb], sems_g[pb])

    def wait_gather(pb):
        pltpu.make_async_copy(h_hbm.at[sidx[0]], rowbufs[pb],
                              sems_g[pb]).wait()

    def issue_scatter(pb, pi):
        pltpu.async_copy(rowbufs[pb], acc_sh.at[didx[pi]], sems_s[pb],
                         add=True)

    def wait_scatter(pb, pi):
        pltpu.make_async_copy(rowbufs[pb], acc_sh.at[didx[pi]],
                              sems_s[pb]).wait()

    def coef_scale(pb, pi):
        buf = rowbufs[pb]
        for g in range(K // L):
            sl = pl.ds(g * L, L)
            e = jnp.maximum(coefb[pi][sl], 1e-5)
            a = plsc.load_gather(dinv, [sidx[pi][sl]])
            bb = plsc.load_gather(dinv, [didx[pi][sl]])
            coefb[pi][sl] = e * a * bb

        for g in range(K // L):
            cvals = coefb[pi][pl.ds(g * L, L)]

            @plsc.parallel_loop(0, L, unroll=2)
            def _(j2):
                cv = cvals[jnp.full((L,), j2, jnp.int32)]
                j = g * L + j2
                for k2 in range(D // L):
                    sl2 = pl.ds(k2 * L, L)
                    buf[j, sl2] = buf[j, sl2] * cv

    def body(i, k, wait_prev_scatter=True, next2=True, next3=True):
        pb, pi = k % NBUF, k % NIDX
        wait_gather(pb)
        coef_scale(pb, pi)
        issue_scatter(pb, pi)
        if wait_prev_scatter:
            wait_scatter((k - 1) % NBUF, (k - 1) % NIDX)
        if next2:
            wait_idx(i + 2, (k + 2) % NIDX)
            issue_gather((k + 2) % NBUF, (k + 2) % NIDX)
        if next3:
            issue_idx(i + 3, (k + 3) % NIDX)

    issue_idx(0, 0)
    issue_idx(1, 1)
    issue_idx(2, 2)
    wait_idx(0, 0)
    issue_gather(0, 0)
    wait_idx(1, 1)
    issue_gather(1, 1)
    dinv_cp.wait()
    body(0, 0, wait_prev_scatter=False)
    body(1, 1)

    @pl.loop(0, (MAIN_CHUNKS - 5) // 12)
    def _(jj):
        for u in range(12):
            body(2 + jj * 12 + u, 2 + u)

    body(MAIN_CHUNKS - 3, MAIN_CHUNKS - 3, next3=False)
    body(MAIN_CHUNKS - 2, MAIN_CHUNKS - 2, next2=False, next3=False)
    body(MAIN_CHUNKS - 1, MAIN_CHUNKS - 1, next2=False, next3=False)
    wait_scatter((MAIN_CHUNKS - 1) % NBUF, (MAIN_CHUNKS - 1) % NIDX)

    plsc.subcore_barrier()

    @pl.when(s < NS - 1)
    def _():
        pltpu.sync_copy(acc_sh.at[pl.ds(s * NSTRIPE, NSTRIPE)],
                        part_hbm.at[c, pl.ds(s * NSTRIPE, NSTRIPE)])

    @pl.when(s == NS - 1)
    def _():
        base = (NS - 1) * NSTRIPE
        pltpu.sync_copy(acc_sh.at[pl.ds(base, LAST_STRIPE)],
                        part_hbm.at[c, pl.ds(base, LAST_STRIPE)])


_sc_gcn = functools.partial(
    pl.kernel,
    out_type=jax.ShapeDtypeStruct((NC, N, D), jnp.float32),
    mesh=_SC_MESH,
    compiler_params=_SC_PARAMS,
    scratch_types=[
        pltpu.VMEM_SHARED((N, D), jnp.float32),
        [pltpu.VMEM((K,), jnp.int32)] * NIDX,
        [pltpu.VMEM((K,), jnp.int32)] * NIDX,
        [pltpu.VMEM((K,), jnp.float32)] * NIDX,
        pltpu.VMEM((N,), jnp.float32),
        [pltpu.VMEM((K, D), jnp.float32)] * NBUF,
        pltpu.SemaphoreType.DMA,
        [pltpu.SemaphoreType.DMA] * NIDX,
        [pltpu.SemaphoreType.DMA] * NBUF,
        [pltpu.SemaphoreType.DMA] * NBUF,
    ],
)(_sc_body)


def _mm_body(x_ref, w_ref, degp_ref, h_ref, dinv_ref):
    h_ref[...] = jnp.dot(x_ref[...], w_ref[...],
                         preferred_element_type=jnp.float32)
    dd = degp_ref[...]
    deg = dd[0:1] + dd[1:2]
    dinv_ref[...] = jnp.where(deg > 0.0,
                              lax.rsqrt(jnp.maximum(deg, 1e-12)), 0.0)


def _comb_body(p_ref, b_ref, o_ref):
    o_ref[...] = p_ref[0] + p_ref[1] + b_ref[...]


def kernel(x, edge_index, edge_weight, W, b):
    src = edge_index[0].astype(jnp.int32)
    dst = edge_index[1].astype(jnp.int32)
    ew = edge_weight.astype(jnp.float32)
    src3 = src.reshape(NC * NS, MAIN_CHUNKS, K)
    dst3 = dst.reshape(NC * NS, MAIN_CHUNKS, K)
    ew3 = ew.reshape(NC * NS, MAIN_CHUNKS, K)
    degp = _sc_deg(dst3, ew3)
    h, dinv = pl.pallas_call(
        _mm_body,
        grid=(10,),
        in_specs=[pl.BlockSpec((N // 10, D), lambda i: (i, 0)),
                  pl.BlockSpec((D, D), lambda i: (0, 0)),
                  pl.BlockSpec((NC, N), lambda i: (0, 0))],
        out_specs=[pl.BlockSpec((N // 10, D), lambda i: (i, 0)),
                   pl.BlockSpec((1, N), lambda i: (0, 0))],
        out_shape=[jax.ShapeDtypeStruct((N, D), jnp.float32),
                   jax.ShapeDtypeStruct((1, N), jnp.float32)],
    )(x, W, degp)
    part = _sc_gcn(src3, dst3, ew3, h, dinv.reshape(N))
    out = pl.pallas_call(
        _comb_body,
        grid=(10,),
        in_specs=[pl.BlockSpec((NC, N // 10, D), lambda i: (0, i, 0)),
                  pl.BlockSpec((1, D), lambda i: (0, 0))],
        out_specs=pl.BlockSpec((N // 10, D), lambda i: (i, 0)),
        out_shape=jax.ShapeDtypeStruct((N, D), jnp.float32),
    )(part, b.reshape(1, D))
    return out


# speedup vs baseline: 1.0609x; 1.0609x over previous
import functools

import jax
import jax.numpy as jnp
from jax import lax
from jax.experimental import pallas as pl
from jax.experimental.pallas import tpu as pltpu
from jax.experimental.pallas import tpu_sc as plsc

N = 10000
E = 320000
D = 128
NC, NS, L = 2, 16, 16
K = 80
E_TILE = E // (NC * NS)
MAIN_CHUNKS = E_TILE // K
NBUF = 3
NIDX = 4
DGB = 5
DEG_BLOCKS = MAIN_CHUNKS // DGB
NSTRIPE = 640
LAST_STRIPE = N - NSTRIPE * (NS - 1)

_SC_PARAMS = pltpu.CompilerParams(needs_layout_passes=False,
                                  use_tc_tiling_on_sc=False)
_SC_MESH = plsc.VectorSubcoreMesh(core_axis_name="c", subcore_axis_name="s")


def _deg_body(dstm_hbm, ewm_hbm, degp_hbm,
              deg_sh, ddst, dewv, z1d, sem_dma, sems_deg):
    c = lax.axis_index("c")
    s = lax.axis_index("s")
    t = c * NS + s
    zero16 = jnp.zeros((L,), jnp.float32)

    @pl.loop(0, 1040 // L)
    def _(i):
        z1d[pl.ds(i * L, L)] = zero16

    @pl.when(s == 0)
    def _():
        for k in range(N // 1000):
            pltpu.sync_copy(z1d.at[pl.ds(0, 1000)],
                            deg_sh.at[pl.ds(k * 1000, 1000)])

    plsc.subcore_barrier()

    def issue_dma(blk, q):
        pltpu.async_copy(dstm_hbm.at[t].at[pl.ds(blk * DGB, DGB)],
                         ddst[q], sem_dma)
        pltpu.async_copy(ewm_hbm.at[t].at[pl.ds(blk * DGB, DGB)],
                         dewv[q], sem_dma)

    def wait_dma(blk, q):
        pltpu.make_async_copy(dstm_hbm.at[t].at[pl.ds(blk * DGB, DGB)],
                              ddst[q], sem_dma).wait()
        pltpu.make_async_copy(ewm_hbm.at[t].at[pl.ds(blk * DGB, DGB)],
                              dewv[q], sem_dma).wait()

    def process(q):
        @pl.loop(0, DGB)
        def _(r):
            for g in range(K // L):
                sl = pl.ds(g * L, L)
                dewv[q][r, sl] = jnp.maximum(dewv[q][r, sl], 1e-5)

        for r in range(DGB):
            pltpu.async_copy(dewv[q].at[r], deg_sh.at[ddst[q].at[r]],
                             sems_deg[q], add=True)

    def drain(q):
        for r in range(DGB):
            pltpu.make_async_copy(dewv[q].at[r], deg_sh.at[ddst[q].at[r]],
                                  sems_deg[q]).wait()

    issue_dma(0, 0)
    for blk in range(DEG_BLOCKS):
        q = blk % 2
        wait_dma(blk, q)
        if blk > 0:
            drain(1 - q)
        if blk + 1 < DEG_BLOCKS:
            issue_dma(blk + 1, 1 - q)
        process(q)
    drain((DEG_BLOCKS - 1) % 2)

    plsc.subcore_barrier()

    @pl.when(s < NS - 1)
    def _():
        pltpu.sync_copy(deg_sh.at[pl.ds(s * NSTRIPE, NSTRIPE)],
                        degp_hbm.at[c, pl.ds(s * NSTRIPE, NSTRIPE)])

    @pl.when(s == NS - 1)
    def _():
        base = (NS - 1) * NSTRIPE
        pltpu.sync_copy(deg_sh.at[pl.ds(base, LAST_STRIPE)],
                        degp_hbm.at[c, pl.ds(base, LAST_STRIPE)])


_sc_deg = functools.partial(
    pl.kernel,
    out_type=jax.ShapeDtypeStruct((NC, N), jnp.float32),
    mesh=_SC_MESH,
    compiler_params=_SC_PARAMS,
    scratch_types=[
        pltpu.VMEM_SHARED((N,), jnp.float32),
        [pltpu.VMEM((DGB, K), jnp.int32)] * 2,
        [pltpu.VMEM((DGB, K), jnp.float32)] * 2,
        pltpu.VMEM((1040,), jnp.float32),
        pltpu.SemaphoreType.DMA,
        [pltpu.SemaphoreType.DMA] * 2,
    ],
)(_deg_body)


def _sc_body(srcm_hbm, dstm_hbm, ewm_hbm, h_hbm, dinv_hbm, part_hbm,
             acc_sh, sidx, didx, coefb, dinv, rowbufs,
             sem_dinv, sems_i, sems_g, sems_s):
    c = lax.axis_index("c")
    s = lax.axis_index("s")
    t = c * NS + s
    zero16 = jnp.zeros((L,), jnp.float32)

    dinv_cp = pltpu.async_copy(dinv_hbm, dinv, sem_dinv)

    zrow = rowbufs[0]

    @pl.loop(0, K)
    def _(r):
        for k in range(D // L):
            zrow[r, pl.ds(k * L, L)] = zero16

    @pl.when(s < NS - 1)
    def _():
        for k in range(NSTRIPE // K):
            pltpu.sync_copy(zrow, acc_sh.at[pl.ds(s * NSTRIPE + k * K, K)])

    @pl.when(s == NS - 1)
    def _():
        for k in range(LAST_STRIPE // K):
            pltpu.sync_copy(zrow,
                            acc_sh.at[pl.ds((NS - 1) * NSTRIPE + k * K, K)])

    plsc.subcore_barrier()

    def issue_idx(i, p):
        pltpu.async_copy(srcm_hbm.at[t].at[i], sidx[p], sems_i[p])
        pltpu.async_copy(dstm_hbm.at[t].at[i], didx[p], sems_i[p])
        pltpu.async_copy(ewm_hbm.at[t].at[i], coefb[p], sems_i[p])

    def wait_idx(i, p):
        for _ in range(3):
            pltpu.make_async_copy(srcm_hbm.at[t].at[i], sidx[p],
                                  sems_i[p]).wait()

    def issue_gather(pb, pi):
        pltpu.async_copy(h_hbm.at[sidx[pi]], rowbufs[pb], sems_g[pb])

    def wait_gather(pb):
        pltpu.make_async_copy(h_hbm.at[sidx[0]], rowbufs[pb],
                              sems_g[pb]).wait()

    def issue_scatter(pb, pi):
        pltpu.async_copy(rowbufs[pb], acc_sh.at[didx[pi]], sems_s[pb],
                         add=True)

    def wait_scatter(pb, pi):
        pltpu.make_async_copy(rowbufs[pb], acc_sh.at[didx[pi]],
                              sems_s[pb]).wait()

    def coef_scale(pb, pi):
        buf = rowbufs[pb]
        for g in range(K // L):
            sl = pl.ds(g * L, L)
            e = jnp.maximum(coefb[pi][sl], 1e-5)
            a = plsc.load_gather(dinv, [sidx[pi][sl]])
            bb = plsc.load_gather(dinv, [didx[pi][sl]])
            coefb[pi][sl] = e * a * bb

        @plsc.parallel_loop(0, K, unroll=2)
        def _(j):
            cv = plsc.load_gather(coefb[pi], [jnp.full((L,), j, jnp.int32)])
            for k2 in range(D // L):
                sl2 = pl.ds(k2 * L, L)
                buf[j, sl2] = buf[j, sl2] * cv

    def body(i, k, wait_prev_scatter=True, next2=True, next3=True):
        pb, pi = k % NBUF, k % NIDX
        wait_gather(pb)
        coef_scale(pb, pi)
        issue_scatter(pb, pi)
        if wait_prev_scatter:
            wait_scatter((k - 1) % NBUF, (k - 1) % NIDX)
        if next2:
            wait_idx(i + 2, (k + 2) % NIDX)
            issue_gather((k + 2) % NBUF, (k + 2) % NIDX)
        if next3:
            issue_idx(i + 3, (k + 3) % NIDX)

    issue_idx(0, 0)
    issue_idx(1, 1)
    issue_idx(2, 2)
    wait_idx(0, 0)
    issue_gather(0, 0)
    wait_idx(1, 1)
    issue_gather(1, 1)
    dinv_cp.wait()
    body(0, 0, wait_prev_scatter=False)
    body(1, 1)

    @pl.loop(0, (MAIN_CHUNKS - 5) // 12)
    def _(jj):
        for u in range(12):
            body(2 + jj * 12 + u, 2 + u)

    body(MAIN_CHUNKS - 3, MAIN_CHUNKS - 3, next3=False)
    body(MAIN_CHUNKS - 2, MAIN_CHUNKS - 2, next2=False, next3=False)
    body(MAIN_CHUNKS - 1, MAIN_CHUNKS - 1, next2=False, next3=False)
    wait_scatter((MAIN_CHUNKS - 1) % NBUF, (MAIN_CHUNKS - 1) % NIDX)

    plsc.subcore_barrier()

    @pl.when(s < NS - 1)
    def _():
        pltpu.sync_copy(acc_sh.at[pl.ds(s * NSTRIPE, NSTRIPE)],
                        part_hbm.at[c, pl.ds(s * NSTRIPE, NSTRIPE)])

    @pl.when(s == NS - 1)
    def _():
        base = (NS - 1) * NSTRIPE
        pltpu.sync_copy(acc_sh.at[pl.ds(base, LAST_STRIPE)],
                        part_hbm.at[c, pl.ds(base, LAST_STRIPE)])


_sc_gcn = functools.partial(
    pl.kernel,
    out_type=jax.ShapeDtypeStruct((NC, N, D), jnp.float32),
    mesh=_SC_MESH,
    compiler_params=_SC_PARAMS,
    scratch_types=[
        pltpu.VMEM_SHARED((N, D), jnp.float32),
        [pltpu.VMEM((K,), jnp.int32)] * NIDX,
        [pltpu.VMEM((K,), jnp.int32)] * NIDX,
        [pltpu.VMEM((K,), jnp.float32)] * NIDX,
        pltpu.VMEM((N,), jnp.float32),
        [pltpu.VMEM((K, D), jnp.float32)] * NBUF,
        pltpu.SemaphoreType.DMA,
        [pltpu.SemaphoreType.DMA] * NIDX,
        [pltpu.SemaphoreType.DMA] * NBUF,
        [pltpu.SemaphoreType.DMA] * NBUF,
    ],
)(_sc_body)


def _mm_body(x_ref, w_ref, degp_ref, h_ref, dinv_ref):
    h_ref[...] = jnp.dot(x_ref[...], w_ref[...],
                         preferred_element_type=jnp.float32)
    dd = degp_ref[...]
    deg = dd[0:1] + dd[1:2]
    dinv_ref[...] = jnp.where(deg > 0.0,
                              lax.rsqrt(jnp.maximum(deg, 1e-12)), 0.0)


def _comb_body(p_ref, b_ref, o_ref):
    o_ref[...] = p_ref[0] + p_ref[1] + b_ref[...]


def kernel(x, edge_index, edge_weight, W, b):
    src = edge_index[0].astype(jnp.int32)
    dst = edge_index[1].astype(jnp.int32)
    ew = edge_weight.astype(jnp.float32)
    src3 = src.reshape(NC * NS, MAIN_CHUNKS, K)
    dst3 = dst.reshape(NC * NS, MAIN_CHUNKS, K)
    ew3 = ew.reshape(NC * NS, MAIN_CHUNKS, K)
    degp = _sc_deg(dst3, ew3)
    h, dinv = pl.pallas_call(
        _mm_body,
        grid=(10,),
        in_specs=[pl.BlockSpec((N // 10, D), lambda i: (i, 0)),
                  pl.BlockSpec((D, D), lambda i: (0, 0)),
                  pl.BlockSpec((NC, N), lambda i: (0, 0))],
        out_specs=[pl.BlockSpec((N // 10, D), lambda i: (i, 0)),
                   pl.BlockSpec((1, N), lambda i: (0, 0))],
        out_shape=[jax.ShapeDtypeStruct((N, D), jnp.float32),
                   jax.ShapeDtypeStruct((1, N), jnp.float32)],
    )(x, W, degp)
    part = _sc_gcn(src3, dst3, ew3, h, dinv.reshape(N))
    out = pl.pallas_call(
        _comb_body,
        grid=(10,),
        in_specs=[pl.BlockSpec((NC, N // 10, D), lambda i: (0, i, 0)),
                  pl.BlockSpec((1, D), lambda i: (0, 0))],
        out_specs=pl.BlockSpec((N // 10, D), lambda i: (i, 0)),
        out_shape=jax.ShapeDtypeStruct((N, D), jnp.float32),
    )(part, b.reshape(1, D))
    return out
